# Initial kernel scaffold; baseline (speedup 1.0000x reference)
import jax
import jax.numpy as jnp
from jax.experimental import pallas as pl


def kernel(item_seq, table0, table1, fusion_weights):
    raise NotImplementedError("write your pallas kernel here")


# speedup vs baseline: 5.2125x; 5.2125x over previous
import functools

import jax
import jax.numpy as jnp
from jax import lax
from jax.experimental import pallas as pl
from jax.experimental.pallas import tpu as pltpu
from jax.experimental.pallas import tpu_sc as plsc

_BASE = 1000
_DIM = 64
_NC = 2
_NS = 16


def _prep_body(fw_ref, t0_ref, t1_ref, out_ref):
    fw = fw_ref[...]
    e = jnp.exp(fw - jnp.max(fw))
    s = jnp.sum(e)
    ci = lax.broadcasted_iota(jnp.int32, (1, 2), 1)
    w0 = jnp.sum(jnp.where(ci == 0, e, 0.0)) / s
    w1 = jnp.sum(jnp.where(ci == 1, e, 0.0)) / s
    out_ref[0:_BASE, :] = t0_ref[...] * w0
    out_ref[_BASE:, :] = t1_ref[...] * w1


def _make_fused_table(table0, table1, fusion_weights, interpret=False):
    return pl.pallas_call(
        _prep_body,
        out_shape=jax.ShapeDtypeStruct((2 * _BASE, _DIM), jnp.float32),
        interpret=interpret,
    )(fusion_weights.reshape(1, 2), table0, table1)


def _build_sc(n_items, chunk, interpret=False):
    nw = _NC * _NS
    per_w = n_items // nw
    assert per_w * nw == n_items and per_w % chunk == 0
    chunks = per_w // chunk
    ng = chunk // 128
    assert ng * 128 == chunk
    mesh = plsc.VectorSubcoreMesh(
        core_axis_name="c", subcore_axis_name="s",
        num_cores=_NC, num_subcores=_NS,
    )

    @functools.partial(
        pl.kernel,
        out_type=jax.ShapeDtypeStruct((n_items, _DIM), jnp.float32),
        mesh=mesh,
        scratch_types=[
            pltpu.VMEM((chunk,), jnp.int32),
            tuple(pltpu.VMEM((128,), jnp.int32) for _ in range(ng)),
            tuple(pltpu.VMEM((128,), jnp.int32) for _ in range(ng)),
            pltpu.VMEM((chunk, _DIM), jnp.float32),
            pltpu.VMEM((chunk, _DIM), jnp.float32),
            pltpu.SemaphoreType.DMA,
            pltpu.SemaphoreType.DMA,
        ],
        compiler_params=pltpu.CompilerParams(use_tc_tiling_on_sc=False),
        interpret=interpret,
    )
    def sc_fn(idx_hbm, fused_hbm, out_hbm, idx_v, i0_refs, i1_refs,
              r0_v, r1_v, sem0, sem1):
        wid = lax.axis_index("s") * _NC + lax.axis_index("c")
        base = wid * per_w

        @pl.loop(0, chunks)
        def _chunk(g):
            cb = base + g * chunk
            pltpu.sync_copy(idx_hbm.at[pl.ds(cb, chunk)], idx_v)
            for k in range(ng):
                for jj in range(8):
                    j = k * 8 + jj
                    v = idx_v[pl.ds(j * 16, 16)]
                    base_v = jnp.full((16,), _BASE, jnp.int32)
                    hi = lax.div(v, base_v)
                    lo = v - hi * base_v
                    i0_refs[k][pl.ds(jj * 16, 16)] = lo
                    i1_refs[k][pl.ds(jj * 16, 16)] = hi + base_v
            cps = [
                pltpu.async_copy(fused_hbm.at[i0_refs[k]],
                                 r0_v.at[pl.ds(k * 128, 128)], sem0)
                for k in range(ng)
            ] + [
                pltpu.async_copy(fused_hbm.at[i1_refs[k]],
                                 r1_v.at[pl.ds(k * 128, 128)], sem1)
                for k in range(ng)
            ]
            for d in cps:
                d.wait()

            def _add(r, carry):
                for k in range(_DIM // 16):
                    sl = pl.ds(k * 16, 16)
                    r0_v[r, sl] = r0_v[r, sl] + r1_v[r, sl]
                return carry
            lax.fori_loop(0, chunk, _add, 0)

            pltpu.sync_copy(r0_v, out_hbm.at[pl.ds(cb, chunk)])

    return sc_fn


def kernel(item_seq, table0, table1, fusion_weights):
    fused = _make_fused_table(table0, table1, fusion_weights)
    n = item_seq.shape[0] * item_seq.shape[1]
    out = _build_sc(n, 512)(item_seq.reshape(n), fused)
    return out.reshape(item_seq.shape + (_DIM,))


# speedup vs baseline: 5.5535x; 1.0654x over previous
import functools

import jax
import jax.numpy as jnp
from jax import lax
from jax.experimental import pallas as pl
from jax.experimental.pallas import tpu as pltpu
from jax.experimental.pallas import tpu_sc as plsc

_BASE = 1000
_DIM = 64
_NC = 2
_NS = 16


def _prep_body(fw_ref, t0_ref, t1_ref, out_ref):
    fw = fw_ref[...]
    e = jnp.exp(fw - jnp.max(fw))
    s = jnp.sum(e)
    ci = lax.broadcasted_iota(jnp.int32, (1, 2), 1)
    w0 = jnp.sum(jnp.where(ci == 0, e, 0.0)) / s
    w1 = jnp.sum(jnp.where(ci == 1, e, 0.0)) / s
    out_ref[0:_BASE, :] = t0_ref[...] * w0
    out_ref[_BASE:, :] = t1_ref[...] * w1


def _make_fused_table(table0, table1, fusion_weights, interpret=False):
    return pl.pallas_call(
        _prep_body,
        out_shape=jax.ShapeDtypeStruct((2 * _BASE, _DIM), jnp.float32),
        interpret=interpret,
    )(fusion_weights.reshape(1, 2), table0, table1)


def _build_sc(n_items, chunk, interpret=False):
    nw = _NC * _NS
    per_w = n_items // nw
    assert per_w * nw == n_items and per_w % chunk == 0
    chunks = per_w // chunk
    ng = chunk // 128
    assert ng * 128 == chunk
    mesh = plsc.VectorSubcoreMesh(
        core_axis_name="c", subcore_axis_name="s",
        num_cores=_NC, num_subcores=_NS,
    )

    @functools.partial(
        pl.kernel,
        out_type=jax.ShapeDtypeStruct((n_items, _DIM), jnp.float32),
        mesh=mesh,
        scratch_types=[
            pltpu.VMEM((chunk,), jnp.int32),
            tuple(pltpu.VMEM((128,), jnp.int32) for _ in range(ng)),
            tuple(pltpu.VMEM((128,), jnp.int32) for _ in range(ng)),
            pltpu.VMEM((chunk, _DIM), jnp.float32),
            pltpu.VMEM((chunk, _DIM), jnp.float32),
            pltpu.SemaphoreType.DMA,
            pltpu.SemaphoreType.DMA,
        ],
        compiler_params=pltpu.CompilerParams(use_tc_tiling_on_sc=False),
        interpret=interpret,
    )
    def sc_fn(idx_hbm, fused_hbm, out_hbm, idx_v, i0_refs, i1_refs,
              r0_v, r1_v, sem0, sem1):
        wid = lax.axis_index("s") * _NC + lax.axis_index("c")
        base = wid * per_w

        @pl.loop(0, chunks)
        def _chunk(g):
            cb = base + g * chunk
            pltpu.sync_copy(idx_hbm.at[pl.ds(cb, chunk)], idx_v)
            for k in range(ng):
                for jj in range(8):
                    j = k * 8 + jj
                    v = idx_v[pl.ds(j * 16, 16)]
                    base_v = jnp.full((16,), _BASE, jnp.int32)
                    hi = lax.div(v, base_v)
                    lo = v - hi * base_v
                    i0_refs[k][pl.ds(jj * 16, 16)] = lo
                    i1_refs[k][pl.ds(jj * 16, 16)] = hi + base_v
            cps = [
                pltpu.async_copy(fused_hbm.at[i0_refs[k]],
                                 r0_v.at[pl.ds(k * 128, 128)], sem0)
                for k in range(ng)
            ]
            for d in cps:
                d.wait()
            cps = [
                pltpu.async_copy(fused_hbm.at[i1_refs[k]],
                                 r0_v.at[pl.ds(k * 128, 128)], sem1,
                                 add=True)
                for k in range(ng)
            ]
            for d in cps:
                d.wait()

            pltpu.sync_copy(r0_v, out_hbm.at[pl.ds(cb, chunk)])

    return sc_fn


def kernel(item_seq, table0, table1, fusion_weights):
    fused = _make_fused_table(table0, table1, fusion_weights)
    n = item_seq.shape[0] * item_seq.shape[1]
    out = _build_sc(n, 512)(item_seq.reshape(n), fused)
    return out.reshape(item_seq.shape + (_DIM,))
